# Initial kernel scaffold; baseline (speedup 1.0000x reference)
import jax
import jax.numpy as jnp
from jax.experimental import pallas as pl


def kernel(x, p, W, gamma, beta):
    raise NotImplementedError("write your pallas kernel here")


# speedup vs baseline: 15.1845x; 15.1845x over previous
import functools

import jax
import jax.numpy as jnp
import numpy as np
from jax import lax
from jax.experimental import pallas as pl
from jax.experimental.pallas import tpu as pltpu
from jax.experimental.pallas import tpu_sc as plsc

B = 4
N = 16384
NPOINT = 1024
K = 16
IN_F = 64
OUT_F = 128

_F0 = np.asarray(jax.random.randint(jax.random.key(42), (B,), 0, N))

_R = 8
_C = N // _R


def _fps_body(px_ref, py_ref, pz_ref, cent_ref, newp_ref, dist_ref):
    px = px_ref[...]
    py = py_ref[...]
    pz = pz_ref[...]
    dist_ref[...] = jnp.full((B, _R, _C), 1e10, jnp.float32)
    row = lax.broadcasted_iota(jnp.int32, (B, _R, _C), 1)
    col = lax.broadcasted_iota(jnp.int32, (B, _R, _C), 2)
    flat = row * _C + col
    bi = lax.broadcasted_iota(jnp.int32, (B, 1, 1), 0)
    f0 = jnp.zeros((B, 1, 1), jnp.int32)
    for b in range(B):
        f0 = jnp.where(bi == b, jnp.int32(int(_F0[b])), f0)

    def body(i, f):
        mask = flat == f
        zero = jnp.zeros((), jnp.float32)
        cx = jnp.sum(jnp.sum(jnp.where(mask, px, zero), axis=2, keepdims=True),
                     axis=1, keepdims=True)
        cy = jnp.sum(jnp.sum(jnp.where(mask, py, zero), axis=2, keepdims=True),
                     axis=1, keepdims=True)
        cz = jnp.sum(jnp.sum(jnp.where(mask, pz, zero), axis=2, keepdims=True),
                     axis=1, keepdims=True)
        newp_ref[:, pl.ds(i, 1), 0:1] = cx
        newp_ref[:, pl.ds(i, 1), 1:2] = cy
        newp_ref[:, pl.ds(i, 1), 2:3] = cz
        for b in range(B):
            cent_ref[pl.ds(i, 1), b:b + 1] = f[b, :, 0:1].reshape(1, 1)
        d = ((px - cx) ** 2 + (py - cy) ** 2) + (pz - cz) ** 2
        dist = jnp.minimum(dist_ref[...], d)
        dist_ref[...] = dist
        m = jnp.max(jnp.max(dist, axis=2, keepdims=True), axis=1, keepdims=True)
        fn = jnp.where(dist == m, flat, N)
        fn = jnp.min(jnp.min(fn, axis=2, keepdims=True), axis=1, keepdims=True)
        return fn

    lax.fori_loop(0, NPOINT, body, f0, unroll=False)


def _fps(p):
    px = p[:, :, 0].reshape(B, _R, _C)
    py = p[:, :, 1].reshape(B, _R, _C)
    pz = p[:, :, 2].reshape(B, _R, _C)
    cent_t, new_p = pl.pallas_call(
        _fps_body,
        out_shape=(
            jax.ShapeDtypeStruct((NPOINT, B), jnp.int32),
            jax.ShapeDtypeStruct((B, NPOINT, 3), jnp.float32),
        ),
        scratch_shapes=[pltpu.VMEM((B, _R, _C), jnp.float32)],
    )(px, py, pz)
    return cent_t.T, new_p


_QT = 128


def _knn_body(pt_ref, newp_ref, out_ref, key_ref):
    px = pt_ref[0, 0:1, :]
    py = pt_ref[0, 1:2, :]
    pz = pt_ref[0, 2:3, :]
    qx = newp_ref[0, :, 0:1]
    qy = newp_ref[0, :, 1:2]
    qz = newp_ref[0, :, 2:3]
    d = ((qx - px) ** 2 + (qy - py) ** 2) + (qz - pz) ** 2
    key_ref[...] = lax.bitcast_convert_type(d, jnp.int32)
    lane = lax.broadcasted_iota(jnp.int32, (_QT, N), 1)
    big = jnp.int32(np.int32(2 ** 31 - 1))
    kp = jnp.full((_QT, 1), -1, jnp.int32)
    ip = jnp.full((_QT, 1), -1, jnp.int32)
    for j in range(K + 1):
        k = key_ref[...]
        valid = (k > kp) | ((k == kp) & (lane > ip))
        ke = jnp.where(valid, k, big)
        m = jnp.min(ke, axis=1, keepdims=True)
        idx = jnp.min(jnp.where(ke == m, lane, N), axis=1, keepdims=True)
        if j > 0:
            out_ref[0, :, j - 1:j] = idx
        kp, ip = m, idx
    del kp, ip


def _knn(p_t, new_p):
    return pl.pallas_call(
        _knn_body,
        grid=(B, NPOINT // _QT),
        in_specs=[
            pl.BlockSpec((1, 3, N), lambda b, t: (b, 0, 0)),
            pl.BlockSpec((1, _QT, 3), lambda b, t: (b, t, 0)),
        ],
        out_specs=pl.BlockSpec((1, _QT, K), lambda b, t: (b, t, 0)),
        out_shape=jax.ShapeDtypeStruct((B, NPOINT, K), jnp.int32),
        scratch_shapes=[pltpu.VMEM((_QT, N), jnp.int32)],
    )(p_t, new_p)


_NROWS = B * NPOINT * (K + 1)
_NW = 32
_PW = _NROWS // _NW
_CH = 544


_GF = 128


def _gather_body(xflat_hbm, idx_hbm, out_hbm, idx_v, rows_v, sem):
    wid = lax.axis_index("s") * 2 + lax.axis_index("c")
    base = wid * _PW
    for c in range(_PW // _CH):
        off = base + c * _CH
        pltpu.sync_copy(idx_hbm.at[pl.ds(off, _CH)], idx_v)
        pltpu.async_copy(xflat_hbm.at[idx_v], rows_v, sem).wait()
        pltpu.sync_copy(rows_v, out_hbm.at[pl.ds(off, _CH)])


def _gather(xflat, idx_all):
    mesh = plsc.VectorSubcoreMesh(core_axis_name="c", subcore_axis_name="s")
    gk = pl.kernel(
        _gather_body,
        out_type=jax.ShapeDtypeStruct((_NROWS, _GF), jnp.float32),
        mesh=mesh,
        scratch_types=[
            pltpu.VMEM((_CH,), jnp.int32),
            pltpu.VMEM((_CH, _GF), jnp.float32),
            pltpu.SemaphoreType.DMA,
        ],
    )
    return gk(xflat, idx_all)


_MT = 128
_NGRID = B * NPOINT // _MT


def _conv_body(g_ref, wt_ref, maxh_ref, s_ref, ss_ref):
    g = g_ref[:, :, :IN_F]
    diffs = (g[:, 0:1, :] - g[:, 1:, :]).reshape(_MT * K, IN_F)
    h = jnp.dot(diffs, wt_ref[...], preferred_element_type=jnp.float32)
    s_ref[0, 0:1, :] = jnp.sum(h, axis=0, keepdims=True)
    ss_ref[0, 0:1, :] = jnp.sum(h * h, axis=0, keepdims=True)
    maxh_ref[...] = jnp.max(h.reshape(_MT, K, OUT_F), axis=1)


def _conv(g, w_t):
    return pl.pallas_call(
        _conv_body,
        grid=(_NGRID,),
        in_specs=[
            pl.BlockSpec((_MT, K + 1, _GF), lambda i: (i, 0, 0)),
            pl.BlockSpec((IN_F, OUT_F), lambda i: (0, 0)),
        ],
        out_specs=(
            pl.BlockSpec((_MT, OUT_F), lambda i: (i, 0)),
            pl.BlockSpec((1, 1, OUT_F), lambda i: (i, 0, 0)),
            pl.BlockSpec((1, 1, OUT_F), lambda i: (i, 0, 0)),
        ),
        out_shape=(
            jax.ShapeDtypeStruct((B * NPOINT, OUT_F), jnp.float32),
            jax.ShapeDtypeStruct((_NGRID, 1, OUT_F), jnp.float32),
            jax.ShapeDtypeStruct((_NGRID, 1, OUT_F), jnp.float32),
        ),
    )(g, w_t)


def _norm_body(mh_ref, s_ref, ss_ref, gamma_ref, beta_ref, out_ref):
    cnt = jnp.float32(B * NPOINT * K)
    s = jnp.sum(s_ref[...], axis=0) / cnt
    ss = jnp.sum(ss_ref[...], axis=0) / cnt
    var = ss - s * s
    inv = lax.rsqrt(var + 1e-5)
    h = (mh_ref[...] - s) * inv * gamma_ref[...] + beta_ref[...]
    out_ref[...] = jnp.maximum(h, 0.0)


def _norm(maxh, s, ss, gamma, beta):
    return pl.pallas_call(
        _norm_body,
        out_shape=jax.ShapeDtypeStruct((B * NPOINT, OUT_F), jnp.float32),
    )(maxh, s, ss, gamma.reshape(1, OUT_F), beta.reshape(1, OUT_F))


def kernel(x, p, W, gamma, beta):
    cent, new_p = _fps(p)
    p_t = jnp.transpose(p, (0, 2, 1))
    knn_idx = _knn(p_t, new_p)
    idx_all = jnp.concatenate([cent[:, :, None], knn_idx], axis=-1)
    idx_all = idx_all + (jnp.arange(B, dtype=jnp.int32) * N)[:, None, None]
    xp = jnp.pad(x.reshape(B * N, IN_F), ((0, 0), (0, _GF - IN_F)))
    g = _gather(xp, idx_all.reshape(-1))
    maxh, s, ss = _conv(g.reshape(B * NPOINT, K + 1, _GF), W.T)
    out = _norm(maxh, s, ss, gamma, beta)
    return out.reshape(B, NPOINT, OUT_F), new_p


# speedup vs baseline: 21.8408x; 1.4384x over previous
import functools

import jax
import jax.numpy as jnp
import numpy as np
from jax import lax
from jax.experimental import pallas as pl
from jax.experimental.pallas import tpu as pltpu
from jax.experimental.pallas import tpu_sc as plsc

B = 4
N = 16384
NPOINT = 1024
K = 16
IN_F = 64
OUT_F = 128

def _threefry2x32(k1, k2, x0, x1):
    def rotl(v, r):
        return ((v << np.uint32(r)) | (v >> np.uint32(32 - r))).astype(np.uint32)
    ks = [np.uint32(k1), np.uint32(k2),
          np.uint32(k1) ^ np.uint32(k2) ^ np.uint32(0x1BD11BDA)]
    x0 = (x0.astype(np.uint32) + ks[0]).astype(np.uint32)
    x1 = (x1.astype(np.uint32) + ks[1]).astype(np.uint32)
    rots = [[13, 15, 26, 6], [17, 29, 16, 24]]
    kr = ks[1:] + ks[:1]
    for i in range(5):
        for r in rots[i % 2]:
            x0 = (x0 + x1).astype(np.uint32)
            x1 = rotl(x1, r)
            x1 = x1 ^ x0
        x0 = (x0 + kr[0]).astype(np.uint32)
        x1 = (x1 + kr[1] + np.uint32(i + 1)).astype(np.uint32)
        kr = kr[1:] + kr[:1]
    return x0, x1


def _f0_vals():
    b1, b2 = _threefry2x32(0, 42, np.array([0, 0]), np.array([0, 1]))
    l1, l2 = _threefry2x32(b1[1], b2[1], np.zeros(B, np.uint32),
                           np.arange(B, dtype=np.uint32))
    return ((l1 ^ l2) % np.uint32(N)).astype(np.int32)

_R = 8
_C = N // _R


def _fps_body(px_ref, py_ref, pz_ref, cent_ref, newp_ref, dist_ref):
    px = px_ref[...]
    py = py_ref[...]
    pz = pz_ref[...]
    dist_ref[...] = jnp.full((B, _R, _C), 1e10, jnp.float32)
    row = lax.broadcasted_iota(jnp.int32, (B, _R, _C), 1)
    col = lax.broadcasted_iota(jnp.int32, (B, _R, _C), 2)
    flat = row * _C + col
    bi = lax.broadcasted_iota(jnp.int32, (B, 1, 1), 0)
    f0 = jnp.zeros((B, 1, 1), jnp.int32)
    f0v = _f0_vals()
    for b in range(B):
        f0 = jnp.where(bi == b, jnp.int32(int(f0v[b])), f0)

    def body(i, f):
        mask = flat == f
        zero = jnp.zeros((), jnp.float32)
        cx = jnp.sum(jnp.sum(jnp.where(mask, px, zero), axis=2, keepdims=True),
                     axis=1, keepdims=True)
        cy = jnp.sum(jnp.sum(jnp.where(mask, py, zero), axis=2, keepdims=True),
                     axis=1, keepdims=True)
        cz = jnp.sum(jnp.sum(jnp.where(mask, pz, zero), axis=2, keepdims=True),
                     axis=1, keepdims=True)
        newp_ref[:, pl.ds(i, 1), 0:1] = cx
        newp_ref[:, pl.ds(i, 1), 1:2] = cy
        newp_ref[:, pl.ds(i, 1), 2:3] = cz
        for b in range(B):
            cent_ref[pl.ds(i, 1), b:b + 1] = f[b, :, 0:1].reshape(1, 1)
        d = ((px - cx) ** 2 + (py - cy) ** 2) + (pz - cz) ** 2
        dist = jnp.minimum(dist_ref[...], d)
        dist_ref[...] = dist
        m = jnp.max(jnp.max(dist, axis=2, keepdims=True), axis=1, keepdims=True)
        fn = jnp.where(dist == m, flat, N)
        fn = jnp.min(jnp.min(fn, axis=2, keepdims=True), axis=1, keepdims=True)
        return fn

    lax.fori_loop(0, NPOINT, body, f0, unroll=False)


def _fps(p):
    px = p[:, :, 0].reshape(B, _R, _C)
    py = p[:, :, 1].reshape(B, _R, _C)
    pz = p[:, :, 2].reshape(B, _R, _C)
    cent_t, new_p = pl.pallas_call(
        _fps_body,
        out_shape=(
            jax.ShapeDtypeStruct((NPOINT, B), jnp.int32),
            jax.ShapeDtypeStruct((B, NPOINT, 3), jnp.float32),
        ),
        scratch_shapes=[pltpu.VMEM((B, _R, _C), jnp.float32)],
    )(px, py, pz)
    return cent_t.T, new_p


_QT = 128
_SEG = 32
_NS = N // _SEG
_NSEL = K + 1
_BIGI = 2 ** 31 - 1


def _knn_seg_body(p_ref, qt_ref, seg_ref):
    px = p_ref[0, :, 0:1].reshape(_NS, _SEG, 1)
    py = p_ref[0, :, 1:2].reshape(_NS, _SEG, 1)
    pz = p_ref[0, :, 2:3].reshape(_NS, _SEG, 1)
    qx = qt_ref[0, 0:1, :].reshape(1, 1, _QT)
    qy = qt_ref[0, 1:2, :].reshape(1, 1, _QT)
    qz = qt_ref[0, 2:3, :].reshape(1, 1, _QT)
    d = ((qx - px) ** 2 + (qy - py) ** 2) + (qz - pz) ** 2
    smin = lax.bitcast_convert_type(jnp.min(d, axis=1), jnp.int32)
    sid = lax.broadcasted_iota(jnp.int32, (_NS, _QT), 0)
    kp = jnp.full((1, _QT), -1, jnp.int32)
    ip = jnp.full((1, _QT), -1, jnp.int32)
    for j in range(_NSEL):
        valid = (smin > kp) | ((smin == kp) & (sid > ip))
        ke = jnp.where(valid, smin, jnp.int32(_BIGI))
        m = jnp.min(ke, axis=0, keepdims=True)
        idx = jnp.min(jnp.where(ke == m, sid, _NS), axis=0, keepdims=True)
        seg_ref[0, 0, j:j + 1, :] = idx
        kp, ip = m, idx
    del kp, ip


def _knn_segments(p, new_pt):
    return pl.pallas_call(
        _knn_seg_body,
        grid=(B, NPOINT // _QT),
        in_specs=[
            pl.BlockSpec((1, N, 3), lambda b, t: (b, 0, 0)),
            pl.BlockSpec((1, 3, _QT), lambda b, t: (b, 0, t)),
        ],
        out_specs=pl.BlockSpec((1, 1, _NSEL, _QT), lambda b, t: (b, t, 0, 0)),
        out_shape=jax.ShapeDtypeStruct((B, NPOINT // _QT, _NSEL, _QT),
                                       jnp.int32),
    )(p, new_pt)


def _knn_final_body(g2_ref, q_ref, sb_ref, out_ref):
    qx = q_ref[:, 0:1]
    qy = q_ref[:, 1:2]
    qz = q_ref[:, 2:3]
    dds, gids = [], []
    for j in range(_NSEL):
        xs = g2_ref[:, j, 0 * _SEG:1 * _SEG]
        ys = g2_ref[:, j, 1 * _SEG:2 * _SEG]
        zs = g2_ref[:, j, 2 * _SEG:3 * _SEG]
        dds.append(((qx - xs) ** 2 + (qy - ys) ** 2) + (qz - zs) ** 2)
        lane = lax.broadcasted_iota(jnp.int32, (_QT, _SEG), 1)
        gids.append(sb_ref[:, j:j + 1] * _SEG + lane)
    dd = lax.bitcast_convert_type(jnp.concatenate(dds, axis=1), jnp.int32)
    gid = jnp.concatenate(gids, axis=1)
    kp = jnp.full((_QT, 1), -1, jnp.int32)
    ip = jnp.full((_QT, 1), -1, jnp.int32)
    for j in range(_NSEL):
        valid = (dd > kp) | ((dd == kp) & (gid > ip))
        ke = jnp.where(valid, dd, jnp.int32(_BIGI))
        m = jnp.min(ke, axis=1, keepdims=True)
        idx = jnp.min(jnp.where(ke == m, gid, jnp.int32(_BIGI)), axis=1, keepdims=True)
        if j > 0:
            out_ref[:, j - 1:j] = idx
        kp, ip = m, idx
    del kp, ip


def _knn_final(g2, newp_flat, sb):
    return pl.pallas_call(
        _knn_final_body,
        grid=(B * NPOINT // _QT,),
        in_specs=[
            pl.BlockSpec((_QT, _NSEL, 4 * _SEG), lambda i: (i, 0, 0)),
            pl.BlockSpec((_QT, 3), lambda i: (i, 0)),
            pl.BlockSpec((_QT, _NSEL), lambda i: (i, 0)),
        ],
        out_specs=pl.BlockSpec((_QT, K), lambda i: (i, 0)),
        out_shape=jax.ShapeDtypeStruct((B * NPOINT, K), jnp.int32),
    )(g2, newp_flat, sb)


_NROWS = B * NPOINT * (K + 1)
_NW = 32
_PW = _NROWS // _NW
_CH = 544


_GF = 128


def _gather_body(xflat_hbm, idx_hbm, out_hbm, idx_v, rows_v, sem):
    wid = lax.axis_index("s") * 2 + lax.axis_index("c")
    base = wid * _PW
    for c in range(_PW // _CH):
        off = base + c * _CH
        pltpu.sync_copy(idx_hbm.at[pl.ds(off, _CH)], idx_v)
        pltpu.async_copy(xflat_hbm.at[idx_v], rows_v, sem).wait()
        pltpu.sync_copy(rows_v, out_hbm.at[pl.ds(off, _CH)])


def _gather(xflat, idx_all):
    mesh = plsc.VectorSubcoreMesh(core_axis_name="c", subcore_axis_name="s")
    gk = pl.kernel(
        _gather_body,
        out_type=jax.ShapeDtypeStruct((_NROWS, _GF), jnp.float32),
        mesh=mesh,
        scratch_types=[
            pltpu.VMEM((_CH,), jnp.int32),
            pltpu.VMEM((_CH, _GF), jnp.float32),
            pltpu.SemaphoreType.DMA,
        ],
    )
    return gk(xflat, idx_all)


_MT = 128
_NGRID = B * NPOINT // _MT


def _conv_body(g_ref, wt_ref, maxh_ref, s_ref, ss_ref):
    g = g_ref[:, :, :IN_F]
    diffs = (g[:, 0:1, :] - g[:, 1:, :]).reshape(_MT * K, IN_F)
    h = jnp.dot(diffs, wt_ref[...], preferred_element_type=jnp.float32)
    s_ref[0, 0:1, :] = jnp.sum(h, axis=0, keepdims=True)
    ss_ref[0, 0:1, :] = jnp.sum(h * h, axis=0, keepdims=True)
    maxh_ref[...] = jnp.max(h.reshape(_MT, K, OUT_F), axis=1)


def _conv(g, w_t):
    return pl.pallas_call(
        _conv_body,
        grid=(_NGRID,),
        in_specs=[
            pl.BlockSpec((_MT, K + 1, _GF), lambda i: (i, 0, 0)),
            pl.BlockSpec((IN_F, OUT_F), lambda i: (0, 0)),
        ],
        out_specs=(
            pl.BlockSpec((_MT, OUT_F), lambda i: (i, 0)),
            pl.BlockSpec((1, 1, OUT_F), lambda i: (i, 0, 0)),
            pl.BlockSpec((1, 1, OUT_F), lambda i: (i, 0, 0)),
        ),
        out_shape=(
            jax.ShapeDtypeStruct((B * NPOINT, OUT_F), jnp.float32),
            jax.ShapeDtypeStruct((_NGRID, 1, OUT_F), jnp.float32),
            jax.ShapeDtypeStruct((_NGRID, 1, OUT_F), jnp.float32),
        ),
    )(g, w_t)


def _norm_body(mh_ref, s_ref, ss_ref, gamma_ref, beta_ref, out_ref):
    cnt = jnp.float32(B * NPOINT * K)
    s = jnp.sum(s_ref[...], axis=0) / cnt
    ss = jnp.sum(ss_ref[...], axis=0) / cnt
    var = ss - s * s
    inv = lax.rsqrt(var + 1e-5)
    h = (mh_ref[...] - s) * inv * gamma_ref[...] + beta_ref[...]
    out_ref[...] = jnp.maximum(h, 0.0)


def _norm(maxh, s, ss, gamma, beta):
    return pl.pallas_call(
        _norm_body,
        out_shape=jax.ShapeDtypeStruct((B * NPOINT, OUT_F), jnp.float32),
    )(maxh, s, ss, gamma.reshape(1, OUT_F), beta.reshape(1, OUT_F))


def kernel(x, p, W, gamma, beta):
    cent, new_p = _fps(p)
    new_pt = jnp.transpose(new_p, (0, 2, 1))
    segs = _knn_segments(p, new_pt)
    sb = jnp.transpose(segs, (0, 1, 3, 2)).reshape(B * NPOINT, _NSEL)
    seg_idx = (sb.reshape(B, NPOINT, _NSEL)
               + (jnp.arange(B, dtype=jnp.int32) * _NS)[:, None, None])
    pseg = jnp.concatenate(
        [p[:, :, 0].reshape(B, _NS, _SEG),
         p[:, :, 1].reshape(B, _NS, _SEG),
         p[:, :, 2].reshape(B, _NS, _SEG),
         jnp.zeros((B, _NS, _SEG), jnp.float32)],
        axis=2).reshape(B * _NS, 4 * _SEG)
    g2 = _gather(pseg, seg_idx.reshape(-1))
    knn_flat = _knn_final(g2.reshape(B * NPOINT, _NSEL, 4 * _SEG),
                          new_p.reshape(B * NPOINT, 3), sb)
    knn_idx = knn_flat.reshape(B, NPOINT, K)
    idx_all = jnp.concatenate([cent[:, :, None], knn_idx], axis=-1)
    idx_all = idx_all + (jnp.arange(B, dtype=jnp.int32) * N)[:, None, None]
    xp = jnp.pad(x.reshape(B * N, IN_F), ((0, 0), (0, _GF - IN_F)))
    g = _gather(xp, idx_all.reshape(-1))
    maxh, s, ss = _conv(g.reshape(B * NPOINT, K + 1, _GF), W.T)
    out = _norm(maxh, s, ss, gamma, beta)
    return out.reshape(B, NPOINT, OUT_F), new_p


# speedup vs baseline: 21.9859x; 1.0066x over previous
import functools

import jax
import jax.numpy as jnp
import numpy as np
from jax import lax
from jax.experimental import pallas as pl
from jax.experimental.pallas import tpu as pltpu
from jax.experimental.pallas import tpu_sc as plsc

B = 4
N = 16384
NPOINT = 1024
K = 16
IN_F = 64
OUT_F = 128

def _threefry2x32(k1, k2, x0, x1):
    def rotl(v, r):
        return ((v << np.uint32(r)) | (v >> np.uint32(32 - r))).astype(np.uint32)
    ks = [np.uint32(k1), np.uint32(k2),
          np.uint32(k1) ^ np.uint32(k2) ^ np.uint32(0x1BD11BDA)]
    x0 = (x0.astype(np.uint32) + ks[0]).astype(np.uint32)
    x1 = (x1.astype(np.uint32) + ks[1]).astype(np.uint32)
    rots = [[13, 15, 26, 6], [17, 29, 16, 24]]
    kr = ks[1:] + ks[:1]
    for i in range(5):
        for r in rots[i % 2]:
            x0 = (x0 + x1).astype(np.uint32)
            x1 = rotl(x1, r)
            x1 = x1 ^ x0
        x0 = (x0 + kr[0]).astype(np.uint32)
        x1 = (x1 + kr[1] + np.uint32(i + 1)).astype(np.uint32)
        kr = kr[1:] + kr[:1]
    return x0, x1


def _f0_vals():
    b1, b2 = _threefry2x32(0, 42, np.array([0, 0]), np.array([0, 1]))
    l1, l2 = _threefry2x32(b1[1], b2[1], np.zeros(B, np.uint32),
                           np.arange(B, dtype=np.uint32))
    return ((l1 ^ l2) % np.uint32(N)).astype(np.int32)

_R = 8
_C = N // _R


def _fps_body(px_ref, py_ref, pz_ref, cent_ref, cpx_ref, cpy_ref, cpz_ref,
              dist_ref):
    px = px_ref[...]
    py = py_ref[...]
    pz = pz_ref[...]
    dist_ref[...] = jnp.full((B, _R, _C), 1e10, jnp.float32)
    row = lax.broadcasted_iota(jnp.int32, (B, _R, _C), 1)
    col = lax.broadcasted_iota(jnp.int32, (B, _R, _C), 2)
    flat = row * _C + col
    arow = lax.broadcasted_iota(jnp.int32, (1, _R, _QT), 1)
    acol = lax.broadcasted_iota(jnp.int32, (1, _R, _QT), 2)
    bi = lax.broadcasted_iota(jnp.int32, (B, 1, 1), 0)
    f0 = jnp.zeros((B, 1, 1), jnp.int32)
    f0v = _f0_vals()
    for b in range(B):
        f0 = jnp.where(bi == b, jnp.int32(int(f0v[b])), f0)
    zi = jnp.zeros((B, _R, _QT), jnp.int32)
    zf = jnp.zeros((B, _R, _QT), jnp.float32)

    def body(i, carry):
        f, cent, cpx, cpy, cpz = carry
        mask = flat == f
        zero = jnp.zeros((), jnp.float32)
        cx = jnp.sum(jnp.sum(jnp.where(mask, px, zero), axis=2, keepdims=True),
                     axis=1, keepdims=True)
        cy = jnp.sum(jnp.sum(jnp.where(mask, py, zero), axis=2, keepdims=True),
                     axis=1, keepdims=True)
        cz = jnp.sum(jnp.sum(jnp.where(mask, pz, zero), axis=2, keepdims=True),
                     axis=1, keepdims=True)
        amask = (arow == (i >> 7)) & (acol == (i & 127))
        cent = jnp.where(amask, f, cent)
        cpx = jnp.where(amask, cx, cpx)
        cpy = jnp.where(amask, cy, cpy)
        cpz = jnp.where(amask, cz, cpz)
        d = ((px - cx) ** 2 + (py - cy) ** 2) + (pz - cz) ** 2
        dist = jnp.minimum(dist_ref[...], d)
        dist_ref[...] = dist
        m = jnp.max(jnp.max(dist, axis=2, keepdims=True), axis=1, keepdims=True)
        fn = jnp.where(dist == m, flat, N)
        fn = jnp.min(jnp.min(fn, axis=2, keepdims=True), axis=1, keepdims=True)
        return (fn, cent, cpx, cpy, cpz)

    _, cent, cpx, cpy, cpz = lax.fori_loop(0, NPOINT, body,
                                           (f0, zi, zf, zf, zf), unroll=2)
    cent_ref[...] = cent
    cpx_ref[...] = cpx
    cpy_ref[...] = cpy
    cpz_ref[...] = cpz


def _fps(p):
    px = p[:, :, 0].reshape(B, _R, _C)
    py = p[:, :, 1].reshape(B, _R, _C)
    pz = p[:, :, 2].reshape(B, _R, _C)
    sh = jax.ShapeDtypeStruct((B, _R, _QT), jnp.float32)
    cent, cpx, cpy, cpz = pl.pallas_call(
        _fps_body,
        out_shape=(
            jax.ShapeDtypeStruct((B, _R, _QT), jnp.int32),
            sh, sh, sh,
        ),
        scratch_shapes=[pltpu.VMEM((B, _R, _C), jnp.float32)],
    )(px, py, pz)
    new_p = jnp.stack([cpx, cpy, cpz], axis=-1).reshape(B, NPOINT, 3)
    return cent.reshape(B, NPOINT), new_p


_QT = 128
_SEG = 32
_NS = N // _SEG
_NSEL = K + 1
_BIGI = 2 ** 31 - 1


def _knn_seg_body(p_ref, qt_ref, seg_ref):
    px = p_ref[0, :, 0:1].reshape(_NS, _SEG, 1)
    py = p_ref[0, :, 1:2].reshape(_NS, _SEG, 1)
    pz = p_ref[0, :, 2:3].reshape(_NS, _SEG, 1)
    qx = qt_ref[0, 0:1, :].reshape(1, 1, _QT)
    qy = qt_ref[0, 1:2, :].reshape(1, 1, _QT)
    qz = qt_ref[0, 2:3, :].reshape(1, 1, _QT)
    d = ((qx - px) ** 2 + (qy - py) ** 2) + (qz - pz) ** 2
    smin = lax.bitcast_convert_type(jnp.min(d, axis=1), jnp.int32)
    sid = lax.broadcasted_iota(jnp.int32, (_NS, _QT), 0)
    kp = jnp.full((1, _QT), -1, jnp.int32)
    ip = jnp.full((1, _QT), -1, jnp.int32)
    for j in range(_NSEL):
        valid = (smin > kp) | ((smin == kp) & (sid > ip))
        ke = jnp.where(valid, smin, jnp.int32(_BIGI))
        m = jnp.min(ke, axis=0, keepdims=True)
        idx = jnp.min(jnp.where(ke == m, sid, _NS), axis=0, keepdims=True)
        seg_ref[0, 0, j:j + 1, :] = idx
        kp, ip = m, idx
    del kp, ip


def _knn_segments(p, new_pt):
    return pl.pallas_call(
        _knn_seg_body,
        grid=(B, NPOINT // _QT),
        in_specs=[
            pl.BlockSpec((1, N, 3), lambda b, t: (b, 0, 0)),
            pl.BlockSpec((1, 3, _QT), lambda b, t: (b, 0, t)),
        ],
        out_specs=pl.BlockSpec((1, 1, _NSEL, _QT), lambda b, t: (b, t, 0, 0)),
        out_shape=jax.ShapeDtypeStruct((B, NPOINT // _QT, _NSEL, _QT),
                                       jnp.int32),
    )(p, new_pt)


def _knn_final_body(g2_ref, q_ref, sb_ref, out_ref):
    qx = q_ref[:, 0:1]
    qy = q_ref[:, 1:2]
    qz = q_ref[:, 2:3]
    dds, gids = [], []
    for j in range(_NSEL):
        xs = g2_ref[:, j, 0 * _SEG:1 * _SEG]
        ys = g2_ref[:, j, 1 * _SEG:2 * _SEG]
        zs = g2_ref[:, j, 2 * _SEG:3 * _SEG]
        dds.append(((qx - xs) ** 2 + (qy - ys) ** 2) + (qz - zs) ** 2)
        lane = lax.broadcasted_iota(jnp.int32, (_QT, _SEG), 1)
        gids.append(sb_ref[:, j:j + 1] * _SEG + lane)
    dd = lax.bitcast_convert_type(jnp.concatenate(dds, axis=1), jnp.int32)
    gid = jnp.concatenate(gids, axis=1)
    kp = jnp.full((_QT, 1), -1, jnp.int32)
    ip = jnp.full((_QT, 1), -1, jnp.int32)
    for j in range(_NSEL):
        valid = (dd > kp) | ((dd == kp) & (gid > ip))
        ke = jnp.where(valid, dd, jnp.int32(_BIGI))
        m = jnp.min(ke, axis=1, keepdims=True)
        idx = jnp.min(jnp.where(ke == m, gid, jnp.int32(_BIGI)), axis=1, keepdims=True)
        if j > 0:
            out_ref[:, j - 1:j] = idx
        kp, ip = m, idx
    del kp, ip


def _knn_final(g2, newp_flat, sb):
    return pl.pallas_call(
        _knn_final_body,
        grid=(B * NPOINT // _QT,),
        in_specs=[
            pl.BlockSpec((_QT, _NSEL, 4 * _SEG), lambda i: (i, 0, 0)),
            pl.BlockSpec((_QT, 3), lambda i: (i, 0)),
            pl.BlockSpec((_QT, _NSEL), lambda i: (i, 0)),
        ],
        out_specs=pl.BlockSpec((_QT, K), lambda i: (i, 0)),
        out_shape=jax.ShapeDtypeStruct((B * NPOINT, K), jnp.int32),
    )(g2, newp_flat, sb)


_NROWS = B * NPOINT * (K + 1)
_NW = 32
_PW = _NROWS // _NW
_CH = 544


_GF = 128


def _gather_body(xflat_hbm, idx_hbm, out_hbm, idx_v, rows_v, sem):
    wid = lax.axis_index("s") * 2 + lax.axis_index("c")
    base = wid * _PW
    for c in range(_PW // _CH):
        off = base + c * _CH
        pltpu.sync_copy(idx_hbm.at[pl.ds(off, _CH)], idx_v)
        pltpu.async_copy(xflat_hbm.at[idx_v], rows_v, sem).wait()
        pltpu.sync_copy(rows_v, out_hbm.at[pl.ds(off, _CH)])


def _gather(xflat, idx_all):
    mesh = plsc.VectorSubcoreMesh(core_axis_name="c", subcore_axis_name="s")
    gk = pl.kernel(
        _gather_body,
        out_type=jax.ShapeDtypeStruct((_NROWS, _GF), jnp.float32),
        mesh=mesh,
        scratch_types=[
            pltpu.VMEM((_CH,), jnp.int32),
            pltpu.VMEM((_CH, _GF), jnp.float32),
            pltpu.SemaphoreType.DMA,
        ],
    )
    return gk(xflat, idx_all)


_MT = 128
_NGRID = B * NPOINT // _MT


def _conv_body(g_ref, par_ref, wt_ref, maxh_ref, s_ref, ss_ref):
    par = par_ref[...][:, :, None]
    g = jnp.where(par == 1, g_ref[:, :, IN_F:], g_ref[:, :, :IN_F])
    diffs = (g[:, 0:1, :] - g[:, 1:, :]).reshape(_MT * K, IN_F)
    h = jnp.dot(diffs, wt_ref[...], preferred_element_type=jnp.float32)
    s_ref[0, 0:1, :] = jnp.sum(h, axis=0, keepdims=True)
    ss_ref[0, 0:1, :] = jnp.sum(h * h, axis=0, keepdims=True)
    maxh_ref[...] = jnp.max(h.reshape(_MT, K, OUT_F), axis=1)


def _conv(g, par, w_t):
    return pl.pallas_call(
        _conv_body,
        grid=(_NGRID,),
        in_specs=[
            pl.BlockSpec((_MT, K + 1, _GF), lambda i: (i, 0, 0)),
            pl.BlockSpec((_MT, K + 1), lambda i: (i, 0)),
            pl.BlockSpec((IN_F, OUT_F), lambda i: (0, 0)),
        ],
        out_specs=(
            pl.BlockSpec((_MT, OUT_F), lambda i: (i, 0)),
            pl.BlockSpec((1, 1, OUT_F), lambda i: (i, 0, 0)),
            pl.BlockSpec((1, 1, OUT_F), lambda i: (i, 0, 0)),
        ),
        out_shape=(
            jax.ShapeDtypeStruct((B * NPOINT, OUT_F), jnp.float32),
            jax.ShapeDtypeStruct((_NGRID, 1, OUT_F), jnp.float32),
            jax.ShapeDtypeStruct((_NGRID, 1, OUT_F), jnp.float32),
        ),
    )(g, par, w_t)


def _norm_body(mh_ref, s_ref, ss_ref, gamma_ref, beta_ref, out_ref):
    cnt = jnp.float32(B * NPOINT * K)
    s = jnp.sum(s_ref[...], axis=0) / cnt
    ss = jnp.sum(ss_ref[...], axis=0) / cnt
    var = ss - s * s
    inv = lax.rsqrt(var + 1e-5)
    h = (mh_ref[...] - s) * inv * gamma_ref[...] + beta_ref[...]
    out_ref[...] = jnp.maximum(h, 0.0)


def _norm(maxh, s, ss, gamma, beta):
    return pl.pallas_call(
        _norm_body,
        out_shape=jax.ShapeDtypeStruct((B * NPOINT, OUT_F), jnp.float32),
    )(maxh, s, ss, gamma.reshape(1, OUT_F), beta.reshape(1, OUT_F))


def kernel(x, p, W, gamma, beta):
    cent, new_p = _fps(p)
    new_pt = jnp.transpose(new_p, (0, 2, 1))
    segs = _knn_segments(p, new_pt)
    sb = jnp.transpose(segs, (0, 1, 3, 2)).reshape(B * NPOINT, _NSEL)
    seg_idx = (sb.reshape(B, NPOINT, _NSEL)
               + (jnp.arange(B, dtype=jnp.int32) * _NS)[:, None, None])
    pseg = jnp.concatenate(
        [p[:, :, 0].reshape(B, _NS, _SEG),
         p[:, :, 1].reshape(B, _NS, _SEG),
         p[:, :, 2].reshape(B, _NS, _SEG),
         jnp.zeros((B, _NS, _SEG), jnp.float32)],
        axis=2).reshape(B * _NS, 4 * _SEG)
    g2 = _gather(pseg, seg_idx.reshape(-1))
    knn_flat = _knn_final(g2.reshape(B * NPOINT, _NSEL, 4 * _SEG),
                          new_p.reshape(B * NPOINT, 3), sb)
    knn_idx = knn_flat.reshape(B, NPOINT, K)
    idx_all = jnp.concatenate([cent[:, :, None], knn_idx], axis=-1)
    idx_all = idx_all + (jnp.arange(B, dtype=jnp.int32) * N)[:, None, None]
    x2 = x.reshape(B * N // 2, 2 * IN_F)
    g = _gather(x2, (idx_all >> 1).reshape(-1))
    par = (idx_all & 1).reshape(B * NPOINT, K + 1)
    maxh, s, ss = _conv(g.reshape(B * NPOINT, K + 1, _GF), par, W.T)
    out = _norm(maxh, s, ss, gamma, beta)
    return out.reshape(B, NPOINT, OUT_F), new_p


# speedup vs baseline: 22.1051x; 1.0054x over previous
import functools

import jax
import jax.numpy as jnp
import numpy as np
from jax import lax
from jax.experimental import pallas as pl
from jax.experimental.pallas import tpu as pltpu
from jax.experimental.pallas import tpu_sc as plsc

B = 4
N = 16384
NPOINT = 1024
K = 16
IN_F = 64
OUT_F = 128

def _threefry2x32(k1, k2, x0, x1):
    def rotl(v, r):
        return ((v << np.uint32(r)) | (v >> np.uint32(32 - r))).astype(np.uint32)
    ks = [np.uint32(k1), np.uint32(k2),
          np.uint32(k1) ^ np.uint32(k2) ^ np.uint32(0x1BD11BDA)]
    x0 = (x0.astype(np.uint32) + ks[0]).astype(np.uint32)
    x1 = (x1.astype(np.uint32) + ks[1]).astype(np.uint32)
    rots = [[13, 15, 26, 6], [17, 29, 16, 24]]
    kr = ks[1:] + ks[:1]
    for i in range(5):
        for r in rots[i % 2]:
            x0 = (x0 + x1).astype(np.uint32)
            x1 = rotl(x1, r)
            x1 = x1 ^ x0
        x0 = (x0 + kr[0]).astype(np.uint32)
        x1 = (x1 + kr[1] + np.uint32(i + 1)).astype(np.uint32)
        kr = kr[1:] + kr[:1]
    return x0, x1


def _f0_vals():
    b1, b2 = _threefry2x32(0, 42, np.array([0, 0]), np.array([0, 1]))
    l1, l2 = _threefry2x32(b1[1], b2[1], np.zeros(B, np.uint32),
                           np.arange(B, dtype=np.uint32))
    return ((l1 ^ l2) % np.uint32(N)).astype(np.int32)

_R = 8
_C = N // _R


def _fps_body(px_ref, py_ref, pz_ref, cent_ref, cpx_ref, cpy_ref, cpz_ref,
              dist_ref):
    px = px_ref[...]
    py = py_ref[...]
    pz = pz_ref[...]
    dist_ref[...] = jnp.full((B, _R, _C), 1e10, jnp.float32)
    row = lax.broadcasted_iota(jnp.int32, (B, _R, _C), 1)
    col = lax.broadcasted_iota(jnp.int32, (B, _R, _C), 2)
    flat = row * _C + col
    arow = lax.broadcasted_iota(jnp.int32, (1, _R, _QT), 1)
    acol = lax.broadcasted_iota(jnp.int32, (1, _R, _QT), 2)
    bi = lax.broadcasted_iota(jnp.int32, (B, 1, 1), 0)
    f0 = jnp.zeros((B, 1, 1), jnp.int32)
    f0v = _f0_vals()
    for b in range(B):
        f0 = jnp.where(bi == b, jnp.int32(int(f0v[b])), f0)
    zi = jnp.zeros((B, _R, _QT), jnp.int32)
    zf = jnp.zeros((B, _R, _QT), jnp.float32)

    def body(i, carry):
        f, cent, cpx, cpy, cpz = carry
        mask = flat == f
        zero = jnp.zeros((), jnp.float32)
        cx = jnp.sum(jnp.sum(jnp.where(mask, px, zero), axis=2, keepdims=True),
                     axis=1, keepdims=True)
        cy = jnp.sum(jnp.sum(jnp.where(mask, py, zero), axis=2, keepdims=True),
                     axis=1, keepdims=True)
        cz = jnp.sum(jnp.sum(jnp.where(mask, pz, zero), axis=2, keepdims=True),
                     axis=1, keepdims=True)
        amask = (arow == (i >> 7)) & (acol == (i & 127))
        cent = jnp.where(amask, f, cent)
        cpx = jnp.where(amask, cx, cpx)
        cpy = jnp.where(amask, cy, cpy)
        cpz = jnp.where(amask, cz, cpz)
        d = ((px - cx) ** 2 + (py - cy) ** 2) + (pz - cz) ** 2
        dist = jnp.minimum(dist_ref[...], d)
        dist_ref[...] = dist
        m = jnp.max(jnp.max(dist, axis=2, keepdims=True), axis=1, keepdims=True)
        fn = jnp.where(dist == m, flat, N)
        fn = jnp.min(jnp.min(fn, axis=2, keepdims=True), axis=1, keepdims=True)
        return (fn, cent, cpx, cpy, cpz)

    _, cent, cpx, cpy, cpz = lax.fori_loop(0, NPOINT, body,
                                           (f0, zi, zf, zf, zf), unroll=4)
    cent_ref[...] = cent
    cpx_ref[...] = cpx
    cpy_ref[...] = cpy
    cpz_ref[...] = cpz


def _fps(p):
    px = p[:, :, 0].reshape(B, _R, _C)
    py = p[:, :, 1].reshape(B, _R, _C)
    pz = p[:, :, 2].reshape(B, _R, _C)
    sh = jax.ShapeDtypeStruct((B, _R, _QT), jnp.float32)
    cent, cpx, cpy, cpz = pl.pallas_call(
        _fps_body,
        out_shape=(
            jax.ShapeDtypeStruct((B, _R, _QT), jnp.int32),
            sh, sh, sh,
        ),
        scratch_shapes=[pltpu.VMEM((B, _R, _C), jnp.float32)],
    )(px, py, pz)
    new_p = jnp.stack([cpx, cpy, cpz], axis=-1).reshape(B, NPOINT, 3)
    return cent.reshape(B, NPOINT), new_p


_QT = 128
_SEG = 32
_NS = N // _SEG
_NSEL = K + 1
_BIGI = 2 ** 31 - 1


def _knn_seg_body(p_ref, qt_ref, seg_ref):
    px = p_ref[0, :, 0:1].reshape(_NS, _SEG, 1)
    py = p_ref[0, :, 1:2].reshape(_NS, _SEG, 1)
    pz = p_ref[0, :, 2:3].reshape(_NS, _SEG, 1)
    qx = qt_ref[0, 0:1, :].reshape(1, 1, _QT)
    qy = qt_ref[0, 1:2, :].reshape(1, 1, _QT)
    qz = qt_ref[0, 2:3, :].reshape(1, 1, _QT)
    d = ((qx - px) ** 2 + (qy - py) ** 2) + (qz - pz) ** 2
    smin = lax.bitcast_convert_type(jnp.min(d, axis=1), jnp.int32)
    sid = lax.broadcasted_iota(jnp.int32, (_NS, _QT), 0)
    kp = jnp.full((1, _QT), -1, jnp.int32)
    ip = jnp.full((1, _QT), -1, jnp.int32)
    for j in range(_NSEL):
        valid = (smin > kp) | ((smin == kp) & (sid > ip))
        ke = jnp.where(valid, smin, jnp.int32(_BIGI))
        m = jnp.min(ke, axis=0, keepdims=True)
        idx = jnp.min(jnp.where(ke == m, sid, _NS), axis=0, keepdims=True)
        seg_ref[0, 0, j:j + 1, :] = idx
        kp, ip = m, idx
    del kp, ip


def _knn_segments(p, new_pt):
    return pl.pallas_call(
        _knn_seg_body,
        grid=(B, NPOINT // _QT),
        in_specs=[
            pl.BlockSpec((1, N, 3), lambda b, t: (b, 0, 0)),
            pl.BlockSpec((1, 3, _QT), lambda b, t: (b, 0, t)),
        ],
        out_specs=pl.BlockSpec((1, 1, _NSEL, _QT), lambda b, t: (b, t, 0, 0)),
        out_shape=jax.ShapeDtypeStruct((B, NPOINT // _QT, _NSEL, _QT),
                                       jnp.int32),
    )(p, new_pt)


def _knn_final_body(g2_ref, q_ref, sb_ref, out_ref):
    qx = q_ref[:, 0:1]
    qy = q_ref[:, 1:2]
    qz = q_ref[:, 2:3]
    dds, gids = [], []
    for j in range(_NSEL):
        xs = g2_ref[:, j, 0 * _SEG:1 * _SEG]
        ys = g2_ref[:, j, 1 * _SEG:2 * _SEG]
        zs = g2_ref[:, j, 2 * _SEG:3 * _SEG]
        dds.append(((qx - xs) ** 2 + (qy - ys) ** 2) + (qz - zs) ** 2)
        lane = lax.broadcasted_iota(jnp.int32, (_QT, _SEG), 1)
        gids.append(sb_ref[:, j:j + 1] * _SEG + lane)
    dd = lax.bitcast_convert_type(jnp.concatenate(dds, axis=1), jnp.int32)
    gid = jnp.concatenate(gids, axis=1)
    kp = jnp.full((_QT, 1), -1, jnp.int32)
    ip = jnp.full((_QT, 1), -1, jnp.int32)
    for j in range(_NSEL):
        valid = (dd > kp) | ((dd == kp) & (gid > ip))
        ke = jnp.where(valid, dd, jnp.int32(_BIGI))
        m = jnp.min(ke, axis=1, keepdims=True)
        idx = jnp.min(jnp.where(ke == m, gid, jnp.int32(_BIGI)), axis=1, keepdims=True)
        if j > 0:
            out_ref[:, j - 1:j] = idx
        kp, ip = m, idx
    del kp, ip


def _knn_final(g2, newp_flat, sb):
    return pl.pallas_call(
        _knn_final_body,
        grid=(B * NPOINT // _QT,),
        in_specs=[
            pl.BlockSpec((_QT, _NSEL, 4 * _SEG), lambda i: (i, 0, 0)),
            pl.BlockSpec((_QT, 3), lambda i: (i, 0)),
            pl.BlockSpec((_QT, _NSEL), lambda i: (i, 0)),
        ],
        out_specs=pl.BlockSpec((_QT, K), lambda i: (i, 0)),
        out_shape=jax.ShapeDtypeStruct((B * NPOINT, K), jnp.int32),
    )(g2, newp_flat, sb)


_NROWS = B * NPOINT * (K + 1)
_NW = 32
_PW = _NROWS // _NW
_CH = 544


_GF = 128


def _gather_body(xflat_hbm, idx_hbm, out_hbm, idx_v, rows_v, sem):
    wid = lax.axis_index("s") * 2 + lax.axis_index("c")
    base = wid * _PW
    for c in range(_PW // _CH):
        off = base + c * _CH
        pltpu.sync_copy(idx_hbm.at[pl.ds(off, _CH)], idx_v)
        pltpu.async_copy(xflat_hbm.at[idx_v], rows_v, sem).wait()
        pltpu.sync_copy(rows_v, out_hbm.at[pl.ds(off, _CH)])


def _gather(xflat, idx_all):
    mesh = plsc.VectorSubcoreMesh(core_axis_name="c", subcore_axis_name="s")
    gk = pl.kernel(
        _gather_body,
        out_type=jax.ShapeDtypeStruct((_NROWS, _GF), jnp.float32),
        mesh=mesh,
        scratch_types=[
            pltpu.VMEM((_CH,), jnp.int32),
            pltpu.VMEM((_CH, _GF), jnp.float32),
            pltpu.SemaphoreType.DMA,
        ],
    )
    return gk(xflat, idx_all)


_MT = 128
_NGRID = B * NPOINT // _MT


def _conv_body(g_ref, par_ref, wt_ref, maxh_ref, s_ref, ss_ref):
    par = par_ref[...][:, :, None]
    g = jnp.where(par == 1, g_ref[:, :, IN_F:], g_ref[:, :, :IN_F])
    diffs = (g[:, 0:1, :] - g[:, 1:, :]).reshape(_MT * K, IN_F)
    h = jnp.dot(diffs, wt_ref[...], preferred_element_type=jnp.float32)
    s_ref[0, 0:1, :] = jnp.sum(h, axis=0, keepdims=True)
    ss_ref[0, 0:1, :] = jnp.sum(h * h, axis=0, keepdims=True)
    maxh_ref[...] = jnp.max(h.reshape(_MT, K, OUT_F), axis=1)


def _conv(g, par, w_t):
    return pl.pallas_call(
        _conv_body,
        grid=(_NGRID,),
        in_specs=[
            pl.BlockSpec((_MT, K + 1, _GF), lambda i: (i, 0, 0)),
            pl.BlockSpec((_MT, K + 1), lambda i: (i, 0)),
            pl.BlockSpec((IN_F, OUT_F), lambda i: (0, 0)),
        ],
        out_specs=(
            pl.BlockSpec((_MT, OUT_F), lambda i: (i, 0)),
            pl.BlockSpec((1, 1, OUT_F), lambda i: (i, 0, 0)),
            pl.BlockSpec((1, 1, OUT_F), lambda i: (i, 0, 0)),
        ),
        out_shape=(
            jax.ShapeDtypeStruct((B * NPOINT, OUT_F), jnp.float32),
            jax.ShapeDtypeStruct((_NGRID, 1, OUT_F), jnp.float32),
            jax.ShapeDtypeStruct((_NGRID, 1, OUT_F), jnp.float32),
        ),
    )(g, par, w_t)


def _norm_body(mh_ref, s_ref, ss_ref, gamma_ref, beta_ref, out_ref):
    cnt = jnp.float32(B * NPOINT * K)
    s = jnp.sum(s_ref[...], axis=0) / cnt
    ss = jnp.sum(ss_ref[...], axis=0) / cnt
    var = ss - s * s
    inv = lax.rsqrt(var + 1e-5)
    h = (mh_ref[...] - s) * inv * gamma_ref[...] + beta_ref[...]
    out_ref[...] = jnp.maximum(h, 0.0)


def _norm(maxh, s, ss, gamma, beta):
    return pl.pallas_call(
        _norm_body,
        out_shape=jax.ShapeDtypeStruct((B * NPOINT, OUT_F), jnp.float32),
    )(maxh, s, ss, gamma.reshape(1, OUT_F), beta.reshape(1, OUT_F))


def kernel(x, p, W, gamma, beta):
    cent, new_p = _fps(p)
    new_pt = jnp.transpose(new_p, (0, 2, 1))
    segs = _knn_segments(p, new_pt)
    sb = jnp.transpose(segs, (0, 1, 3, 2)).reshape(B * NPOINT, _NSEL)
    seg_idx = (sb.reshape(B, NPOINT, _NSEL)
               + (jnp.arange(B, dtype=jnp.int32) * _NS)[:, None, None])
    pseg = jnp.concatenate(
        [p[:, :, 0].reshape(B, _NS, _SEG),
         p[:, :, 1].reshape(B, _NS, _SEG),
         p[:, :, 2].reshape(B, _NS, _SEG),
         jnp.zeros((B, _NS, _SEG), jnp.float32)],
        axis=2).reshape(B * _NS, 4 * _SEG)
    g2 = _gather(pseg, seg_idx.reshape(-1))
    knn_flat = _knn_final(g2.reshape(B * NPOINT, _NSEL, 4 * _SEG),
                          new_p.reshape(B * NPOINT, 3), sb)
    knn_idx = knn_flat.reshape(B, NPOINT, K)
    idx_all = jnp.concatenate([cent[:, :, None], knn_idx], axis=-1)
    idx_all = idx_all + (jnp.arange(B, dtype=jnp.int32) * N)[:, None, None]
    x2 = x.reshape(B * N // 2, 2 * IN_F)
    g = _gather(x2, (idx_all >> 1).reshape(-1))
    par = (idx_all & 1).reshape(B * NPOINT, K + 1)
    maxh, s, ss = _conv(g.reshape(B * NPOINT, K + 1, _GF), par, W.T)
    out = _norm(maxh, s, ss, gamma, beta)
    return out.reshape(B, NPOINT, OUT_F), new_p
